# baseline (speedup 1.0000x reference)
import jax
import jax.numpy as jnp
from jax import lax
from jax.experimental import pallas as pl
from jax.experimental.pallas import tpu as pltpu
from jax.experimental.pallas import tpu_sc as plsc

BATCH = 16384
HIST = 200
D_MODEL = 32
NC = 2
NS = 16
NW = NC * NS
BH = BATCH * HIST
PER_W = BH // NW
CHUNK = 512
NCHUNK = PER_W // CHUNK
NBUF = 4


def _body(data_hbm, table_hbm, out_hbm, idx_v, rows_v, gsem):
    wid = lax.axis_index("s") * NC + lax.axis_index("c")
    base = wid * PER_W

    def stage(j, b):
        off = base + j * CHUNK
        pltpu.sync_copy(data_hbm.at[pl.ds(off, CHUNK)], idx_v.at[b])
        pltpu.async_copy(table_hbm.at[idx_v.at[b]], rows_v.at[b], gsem.at[b])

    for b in range(NBUF):
        stage(b, b)

    def group(g, carry):
        j0 = g * NBUF
        for b in range(NBUF):
            j = j0 + b
            off = base + j * CHUNK
            pltpu.make_async_copy(
                table_hbm.at[idx_v.at[b]], rows_v.at[b], gsem.at[b]
            ).wait()
            pltpu.sync_copy(rows_v.at[b], out_hbm.at[pl.ds(off, CHUNK)])
            nxt = j + NBUF

            @pl.when(nxt < NCHUNK)
            def _():
                stage(nxt, b)

        return carry

    lax.fori_loop(0, NCHUNK // NBUF, group, 0)


def kernel(data, table):
    idx = data.reshape(BH).astype(jnp.int32)
    mesh = plsc.VectorSubcoreMesh(core_axis_name="c", subcore_axis_name="s")
    out = pl.kernel(
        _body,
        out_type=jax.ShapeDtypeStruct((BH, D_MODEL), jnp.float32),
        mesh=mesh,
        scratch_types=[
            pltpu.VMEM((NBUF, CHUNK), jnp.int32),
            pltpu.VMEM((NBUF, CHUNK, D_MODEL), jnp.float32),
            pltpu.SemaphoreType.DMA((NBUF,)),
        ],
        compiler_params=pltpu.CompilerParams(use_tc_tiling_on_sc=False),
    )(idx, table)
    return out.reshape(BATCH, HIST, D_MODEL)


# speedup vs baseline: 1.0258x; 1.0258x over previous
import jax
import jax.numpy as jnp
from jax import lax
from jax.experimental import pallas as pl
from jax.experimental.pallas import tpu as pltpu
from jax.experimental.pallas import tpu_sc as plsc

BATCH = 16384
HIST = 200
D_MODEL = 32
NC = 2
NS = 16
NW = NC * NS
BH = BATCH * HIST
PER_W = BH // NW
CHUNK = 400
NCHUNK = PER_W // CHUNK
NBUF = 2


def _body(data_hbm, table_hbm, out_hbm, idx_v, rows_v, gsem):
    wid = lax.axis_index("s") * NC + lax.axis_index("c")
    base = wid * PER_W

    pltpu.sync_copy(data_hbm.at[pl.ds(base, PER_W)], idx_v)

    def stage(j, b):
        pltpu.async_copy(
            table_hbm.at[idx_v.at[pl.ds(j * CHUNK, CHUNK)]], rows_v.at[b],
            gsem.at[b],
        )

    for b in range(NBUF):
        stage(b, b)

    def group(g, carry):
        j0 = g * NBUF
        for b in range(NBUF):
            j = j0 + b
            off = base + j * CHUNK
            pltpu.make_async_copy(
                table_hbm.at[idx_v.at[pl.ds(j * CHUNK, CHUNK)]], rows_v.at[b],
                gsem.at[b],
            ).wait()
            pltpu.sync_copy(rows_v.at[b], out_hbm.at[pl.ds(off, CHUNK)])
            nxt = j + NBUF

            @pl.when(nxt < NCHUNK)
            def _():
                stage(nxt, b)

        return carry

    lax.fori_loop(0, NCHUNK // NBUF, group, 0)


def kernel(data, table):
    idx = data.reshape(BH).astype(jnp.int32)
    mesh = plsc.VectorSubcoreMesh(core_axis_name="c", subcore_axis_name="s")
    out = pl.kernel(
        _body,
        out_type=jax.ShapeDtypeStruct((BH, D_MODEL), jnp.float32),
        mesh=mesh,
        scratch_types=[
            pltpu.VMEM((PER_W,), jnp.int32),
            pltpu.VMEM((NBUF, CHUNK, D_MODEL), jnp.float32),
            pltpu.SemaphoreType.DMA((NBUF,)),
        ],
        compiler_params=pltpu.CompilerParams(use_tc_tiling_on_sc=False),
    )(idx, table)
    return out.reshape(BATCH, HIST, D_MODEL)


# speedup vs baseline: 1.0316x; 1.0056x over previous
import jax
import jax.numpy as jnp
from jax import lax
from jax.experimental import pallas as pl
from jax.experimental.pallas import tpu as pltpu
from jax.experimental.pallas import tpu_sc as plsc

BATCH = 16384
HIST = 200
D_MODEL = 32
NC = 2
NS = 16
NW = NC * NS
BH = BATCH * HIST
PER_W = BH // NW
CHUNK = 800
NCHUNK = PER_W // CHUNK
NBUF = 2
NIB = 4


def _body(data_hbm, table_hbm, out_hbm, idx_v, rows_v, isem, gsem):
    wid = lax.axis_index("s") * NC + lax.axis_index("c")
    base = wid * PER_W

    def stage_idx(j, q):
        pltpu.async_copy(
            data_hbm.at[pl.ds(base + j * CHUNK, CHUNK)], idx_v.at[q],
            isem.at[q],
        )

    def stage_gather(j, q, b):
        pltpu.make_async_copy(
            data_hbm.at[pl.ds(base + j * CHUNK, CHUNK)], idx_v.at[q],
            isem.at[q],
        ).wait()
        pltpu.async_copy(
            table_hbm.at[idx_v.at[q]], rows_v.at[b], gsem.at[b]
        )

    for q in range(NIB):
        stage_idx(q, q)
    for b in range(NBUF):
        stage_gather(b, b, b)

    def group(g, carry):
        j0 = g * NIB
        for u in range(NIB):
            j = j0 + u
            b = u % NBUF
            q = u % NIB
            pltpu.make_async_copy(
                table_hbm.at[idx_v.at[q]], rows_v.at[b], gsem.at[b]
            ).wait()
            pltpu.sync_copy(
                rows_v.at[b], out_hbm.at[pl.ds(base + j * CHUNK, CHUNK)]
            )
            ni = j + NIB

            @pl.when(ni < NCHUNK)
            def _():
                stage_idx(ni, q)

            ng = j + NBUF

            @pl.when(ng < NCHUNK)
            def _():
                stage_gather(ng, (ng % NIB), b)

        return carry

    lax.fori_loop(0, NCHUNK // NIB, group, 0)


def kernel(data, table):
    idx = data.reshape(BH).astype(jnp.int32)
    mesh = plsc.VectorSubcoreMesh(core_axis_name="c", subcore_axis_name="s")
    out = pl.kernel(
        _body,
        out_type=jax.ShapeDtypeStruct((BH, D_MODEL), jnp.float32),
        mesh=mesh,
        scratch_types=[
            pltpu.VMEM((NIB, CHUNK), jnp.int32),
            pltpu.VMEM((NBUF, CHUNK, D_MODEL), jnp.float32),
            pltpu.SemaphoreType.DMA((NIB,)),
            pltpu.SemaphoreType.DMA((NBUF,)),
        ],
        compiler_params=pltpu.CompilerParams(use_tc_tiling_on_sc=False),
    )(idx, table)
    return out.reshape(BATCH, HIST, D_MODEL)


# speedup vs baseline: 1.0320x; 1.0004x over previous
import jax
import jax.numpy as jnp
from jax import lax
from jax.experimental import pallas as pl
from jax.experimental.pallas import tpu as pltpu
from jax.experimental.pallas import tpu_sc as plsc

BATCH = 16384
HIST = 200
D_MODEL = 32
NC = 2
NS = 16
NW = NC * NS
BH = BATCH * HIST
PER_W = BH // NW
CHUNK = 800
NCHUNK = PER_W // CHUNK
NBUF = 4
GLEAD = 2
NIB = 4


def _body(data_hbm, table_hbm, out_hbm, idx_v, rows_v, isem, gsem, wsem):
    wid = lax.axis_index("s") * NC + lax.axis_index("c")
    base = wid * PER_W

    def stage_idx(j, q):
        pltpu.async_copy(
            data_hbm.at[pl.ds(base + j * CHUNK, CHUNK)], idx_v.at[q],
            isem.at[q],
        )

    def stage_gather(j, q, b, guard_write):
        if guard_write:

            @pl.when(j >= NBUF)
            def _():
                pltpu.make_async_copy(
                    rows_v.at[b],
                    out_hbm.at[pl.ds(base + (j - NBUF) * CHUNK, CHUNK)],
                    wsem.at[b],
                ).wait()

        pltpu.make_async_copy(
            data_hbm.at[pl.ds(base + j * CHUNK, CHUNK)], idx_v.at[q],
            isem.at[q],
        ).wait()
        pltpu.async_copy(
            table_hbm.at[idx_v.at[q]], rows_v.at[b], gsem.at[b]
        )

    for q in range(NIB):
        stage_idx(q, q)
    for b in range(GLEAD):
        stage_gather(b, b, b, False)

    def group(g, carry):
        j0 = g * NIB
        for u in range(NIB):
            j = j0 + u
            b = u % NBUF
            q = u % NIB
            pltpu.make_async_copy(
                table_hbm.at[idx_v.at[q]], rows_v.at[b], gsem.at[b]
            ).wait()
            pltpu.async_copy(
                rows_v.at[b], out_hbm.at[pl.ds(base + j * CHUNK, CHUNK)],
                wsem.at[b],
            )
            ni = j + NIB

            @pl.when(ni < NCHUNK)
            def _():
                stage_idx(ni, q)

            ng = j + GLEAD

            @pl.when(ng < NCHUNK)
            def _():
                stage_gather(ng, (u + GLEAD) % NIB, (u + GLEAD) % NBUF, True)

        return carry

    lax.fori_loop(0, NCHUNK // NIB, group, 0)

    for u in range(NBUF):
        j = NCHUNK - NBUF + u
        pltpu.make_async_copy(
            rows_v.at[j % NBUF], out_hbm.at[pl.ds(base + j * CHUNK, CHUNK)],
            wsem.at[j % NBUF],
        ).wait()


def kernel(data, table):
    idx = data.reshape(BH).astype(jnp.int32)
    mesh = plsc.VectorSubcoreMesh(core_axis_name="c", subcore_axis_name="s")
    out = pl.kernel(
        _body,
        out_type=jax.ShapeDtypeStruct((BH, D_MODEL), jnp.float32),
        mesh=mesh,
        scratch_types=[
            pltpu.VMEM((NIB, CHUNK), jnp.int32),
            pltpu.VMEM((NBUF, CHUNK, D_MODEL), jnp.float32),
            pltpu.SemaphoreType.DMA((NIB,)),
            pltpu.SemaphoreType.DMA((NBUF,)),
            pltpu.SemaphoreType.DMA((NBUF,)),
        ],
        compiler_params=pltpu.CompilerParams(use_tc_tiling_on_sc=False),
    )(idx, table)
    return out.reshape(BATCH, HIST, D_MODEL)


# speedup vs baseline: 1.0326x; 1.0006x over previous
import jax
import jax.numpy as jnp
from jax import lax
from jax.experimental import pallas as pl
from jax.experimental.pallas import tpu as pltpu
from jax.experimental.pallas import tpu_sc as plsc

BATCH = 16384
HIST = 200
D_MODEL = 32
NC = 2
NS = 16
NW = NC * NS
BH = BATCH * HIST
PER_W = BH // NW
CHUNK = 800
NCHUNK = PER_W // CHUNK
NBUF = 4
GLEAD = 2
NIB = 4


def _body(data_hbm, table_hbm, out_hbm, idx_v, rows_v, isem, gsem, wsem):
    wid = lax.axis_index("s") * NC + lax.axis_index("c")
    base = wid * PER_W

    def stage_idx(j, q):
        pltpu.async_copy(
            data_hbm.at[pl.ds(base + j * CHUNK, CHUNK)], idx_v.at[q],
            isem.at[q],
        )

    def stage_gather(j, q, b, guard_write):
        if guard_write:

            @pl.when(j >= NBUF)
            def _():
                pltpu.make_async_copy(
                    rows_v.at[b],
                    out_hbm.at[pl.ds(base + (j - NBUF) * CHUNK, CHUNK)],
                    wsem.at[b],
                ).wait()

        pltpu.make_async_copy(
            data_hbm.at[pl.ds(base + j * CHUNK, CHUNK)], idx_v.at[q],
            isem.at[q],
        ).wait()
        pltpu.async_copy(
            table_hbm.at[idx_v.at[q]], rows_v.at[b], gsem.at[b]
        )

    for q in range(NIB):
        stage_idx(q, q)
    for b in range(GLEAD):
        stage_gather(b, b, b, False)

    def group(g, carry):
        j0 = g * NIB
        for u in range(NIB):
            j = j0 + u
            b = u % NBUF
            q = u % NIB
            pltpu.make_async_copy(
                table_hbm.at[idx_v.at[q]], rows_v.at[b], gsem.at[b]
            ).wait()
            pltpu.async_copy(
                rows_v.at[b], out_hbm.at[pl.ds(base + j * CHUNK, CHUNK)],
                wsem.at[b],
            )
            ni = j + NIB

            @pl.when(ni < NCHUNK)
            def _():
                stage_idx(ni, q)

            ng = j + GLEAD

            @pl.when(ng < NCHUNK)
            def _():
                stage_gather(ng, (u + GLEAD) % NIB, (u + GLEAD) % NBUF, True)

        return carry

    lax.fori_loop(0, NCHUNK // NIB, group, 0)

    for u in range(NBUF):
        j = NCHUNK - NBUF + u
        pltpu.make_async_copy(
            rows_v.at[j % NBUF], out_hbm.at[pl.ds(base + j * CHUNK, CHUNK)],
            wsem.at[j % NBUF],
        ).wait()


def kernel(data, table):
    idx = data.reshape(BH).astype(jnp.int32)
    mesh = plsc.VectorSubcoreMesh(core_axis_name="c", subcore_axis_name="s")
    out = pl.kernel(
        _body,
        out_type=jax.ShapeDtypeStruct((BH, D_MODEL), jnp.float32),
        mesh=mesh,
        scratch_types=[
            pltpu.VMEM((NIB, CHUNK), jnp.int32),
            pltpu.VMEM((NBUF, CHUNK, D_MODEL), jnp.float32),
            pltpu.SemaphoreType.DMA((NIB,)),
            pltpu.SemaphoreType.DMA((NBUF,)),
            pltpu.SemaphoreType.DMA((NBUF,)),
        ],
        compiler_params=pltpu.CompilerParams(use_tc_tiling_on_sc=False, needs_layout_passes=False),
    )(idx, table)
    return out.reshape(BATCH, HIST, D_MODEL)
